# baseline (device time: 89032 ns/iter reference)
import jax
import jax.numpy as jnp
from jax import lax
from jax.experimental import pallas as pl
from jax.experimental.pallas import tpu as pltpu

N_DEV = 4
F8 = jnp.float8_e5m2


def kernel(x, w_mat, scale_x, scale_w):
    m, k = x.shape
    _k2, n = w_mat.shape
    m_out = m // N_DEV
    kh = k // 2

    s = (scale_x * scale_w).reshape(1, 1).astype(jnp.float32)

    def body(x_hbm, w_hbm, s_ref, out_ref,
             w_all, xg, x8s, w8s, x_ref, w_ref,
             copy_sems,
             xs_sems, xr_sems,
             cws_sems, cwr_sems, ccws_sems, ccwr_sems):
        w_cp = pltpu.make_async_copy(w_hbm, w_ref, copy_sems.at[0])
        x_cp = pltpu.make_async_copy(x_hbm, x_ref, copy_sems.at[1])
        w_cp.start()
        x_cp.start()
        my = lax.axis_index("i")
        left = lax.rem(my + N_DEV - 1, N_DEV)
        right = lax.rem(my + 1, N_DEV)
        diag = lax.rem(my + 2, N_DEV)

        barrier_sem = pltpu.get_barrier_semaphore()
        for nbr in (left, right, diag):
            pl.semaphore_signal(
                barrier_sem, inc=1,
                device_id=(nbr,), device_id_type=pl.DeviceIdType.MESH,
            )
        pl.semaphore_wait(barrier_sem, N_DEV - 1)

        kq = kh // 2

        def w_slice(c, half):
            return pl.ds(c * k + half * kh, kh)

        def wq_slice(c, half, q):
            return pl.ds(c * k + half * kh + q * kq, kq)

        def rdma(src, dst, ssem, rsem, dev):
            return pltpu.make_async_remote_copy(
                src_ref=src, dst_ref=dst, send_sem=ssem, recv_sem=rsem,
                device_id=(dev,), device_id_type=pl.DeviceIdType.MESH,
            )

        ct = 256
        w_cp.wait()
        for t in range(k // ct):
            r = pl.ds(t * ct, ct)
            w8s[r, :] = w_ref[r, :].astype(F8)
        def w_rdma(c, half, q, hop, dev):
            sems = (cws_sems, cwr_sems) if half == 0 else (ccws_sems, ccwr_sems)
            return rdma(
                (w8s if hop == 0 else w_all).at[
                    (pl.ds(half * kh + q * kq, kq) if hop == 0
                     else wq_slice(c, half, q)), :],
                w_all.at[wq_slice(c, half, q), :],
                sems[0].at[2 * hop + q], sems[1].at[2 * hop + q], dev)

        cw0 = [w_rdma(my, 0, q, 0, right) for q in range(2)]
        ccw0 = [w_rdma(my, 1, q, 0, left) for q in range(2)]
        for d in cw0 + ccw0:
            d.start()

        x_cp.wait()
        for t in range(m // ct):
            r = pl.ds(t * ct, ct)
            x8s[r, :] = x_ref[r, :].astype(F8)
        xd2 = rdma(x8s.at[pl.ds(diag * m_out, m_out), :], xg.at[2],
                   xs_sems.at[2], xr_sems.at[2], diag)
        xd0 = rdma(x8s.at[pl.ds(right * m_out, m_out), :], xg.at[0],
                   xs_sems.at[0], xr_sems.at[0], right)
        xd1 = rdma(x8s.at[pl.ds(left * m_out, m_out), :], xg.at[1],
                   xs_sems.at[1], xr_sems.at[1], left)
        for d in cw0 + ccw0:
            d.wait_send()
        for d in (xd2, xd0, xd1):
            d.start()

        mt = 256
        n_tiles = m_out // mt

        def add_gathered_panel(xslot, c, half):
            wv = w_all[w_slice(c, half), :]
            for t in range(n_tiles):
                r = pl.ds(t * mt, mt)
                xv = xg[xslot, t * mt:(t + 1) * mt, half * kh:(half + 1) * kh]
                out_ref[r, :] = out_ref[r, :] + jnp.dot(
                    xv, wv, preferred_element_type=jnp.float32)

        wv0 = w8s[:, :]
        for t in range(n_tiles):
            r = pl.ds(my * m_out + t * mt, mt)
            out_ref[pl.ds(t * mt, mt), :] = jnp.dot(
                x8s[r, :], wv0, preferred_element_type=jnp.float32)

        cw1 = [w_rdma(left, 0, q, 1, right) for q in range(2)]
        ccw1 = [w_rdma(right, 1, q, 1, left) for q in range(2)]
        cw2 = [w_rdma(diag, 0, q, 2, right) for q in range(2)]
        ccw2 = [w_rdma(diag, 1, q, 2, left) for q in range(2)]
        for q in range(2):
            cw0[q].wait_recv()
            cw1[q].start()
            ccw0[q].wait_recv()
            ccw1[q].start()
        for q in range(2):
            cw1[q].wait_recv()
            cw2[q].start()
            ccw1[q].wait_recv()
            ccw2[q].start()

        xd0.wait_recv()
        add_gathered_panel(0, left, 0)
        xd1.wait_recv()
        add_gathered_panel(1, right, 1)
        xd2.wait_recv()
        add_gathered_panel(2, diag, 0)
        add_gathered_panel(2, diag, 1)

        def add_quarter_panel(xslot, c, half, q):
            wv = w_all[wq_slice(c, half, q), :]
            for t in range(n_tiles):
                r = pl.ds(t * mt, mt)
                xv = xg[xslot, t * mt:(t + 1) * mt,
                        half * kh + q * kq:half * kh + (q + 1) * kq]
                out_ref[r, :] = out_ref[r, :] + jnp.dot(
                    xv, wv, preferred_element_type=jnp.float32)

        cw2[0].wait_recv()
        add_quarter_panel(1, right, 0, 0)
        ccw2[0].wait_recv()
        add_quarter_panel(0, left, 1, 0)
        cw2[1].wait_recv()
        add_quarter_panel(1, right, 0, 1)

        for d in [xd0, xd1, xd2] + cw1 + ccw1 + cw2:
            d.wait_send()

        ccw2[1].wait_recv()
        sc = s_ref[0, 0]
        wv_last = w_all[wq_slice(left, 1, 1), :]
        for t in range(n_tiles):
            r = pl.ds(t * mt, mt)
            xv = xg[0, t * mt:(t + 1) * mt, kh + kq:kh + 2 * kq]
            y = (out_ref[r, :] + jnp.dot(
                xv, wv_last, preferred_element_type=jnp.float32)) * sc
            out_ref[r, :] = y * jax.nn.sigmoid(y)
        for d in ccw2:
            d.wait_send()

        def _exit_barrier(exit_sem):
            for nbr in (left, right, diag):
                pl.semaphore_signal(
                    exit_sem, inc=1,
                    device_id=(nbr,), device_id_type=pl.DeviceIdType.MESH,
                )
            pl.semaphore_wait(exit_sem, N_DEV - 1)

        pl.run_scoped(_exit_barrier, exit_sem=pltpu.SemaphoreType.REGULAR)

    return pl.pallas_call(
        body,
        out_shape=jax.ShapeDtypeStruct((m_out, n), jnp.float32),
        in_specs=[
            pl.BlockSpec(memory_space=pltpu.MemorySpace.HBM),
            pl.BlockSpec(memory_space=pltpu.MemorySpace.HBM),
            pl.BlockSpec(memory_space=pltpu.SMEM),
        ],
        out_specs=pl.BlockSpec(memory_space=pltpu.VMEM),
        scratch_shapes=[
            pltpu.VMEM((N_DEV * k, n), F8),
            pltpu.VMEM((3, m_out, k), F8),
            pltpu.VMEM((m, k), F8),
            pltpu.VMEM((k, n), F8),
            pltpu.VMEM((m, k), jnp.float32),
            pltpu.VMEM((k, n), jnp.float32),
            pltpu.SemaphoreType.DMA((2,)),
            pltpu.SemaphoreType.DMA((3,)),
            pltpu.SemaphoreType.DMA((3,)),
            pltpu.SemaphoreType.DMA((6,)),
            pltpu.SemaphoreType.DMA((6,)),
            pltpu.SemaphoreType.DMA((6,)),
            pltpu.SemaphoreType.DMA((6,)),
        ],
        compiler_params=pltpu.CompilerParams(
            collective_id=0,
            vmem_limit_bytes=100 * 1024 * 1024,
        ),
    )(x, w_mat, s)


# device time: 78356 ns/iter; 1.1362x vs baseline; 1.1362x over previous
import jax
import jax.numpy as jnp
from jax import lax
from jax.experimental import pallas as pl
from jax.experimental.pallas import tpu as pltpu

N_DEV = 4
F8 = jnp.float8_e5m2


def kernel(x, w_mat, scale_x, scale_w):
    m, k = x.shape
    _k2, n = w_mat.shape
    m_out = m // N_DEV
    kh = k // 2

    s = (scale_x * scale_w).reshape(1, 1).astype(jnp.float32)

    def body(x_hbm, w_hbm, s_ref, out_ref,
             w_all, xg, x8s, w8s, x_ref, w_ref,
             copy_sems,
             xs_sems, xr_sems,
             cws_sems, cwr_sems, ccws_sems, ccwr_sems):
        w_cp = pltpu.make_async_copy(w_hbm, w_ref, copy_sems.at[0])
        x_cp = pltpu.make_async_copy(x_hbm, x_ref, copy_sems.at[1])
        w_cp.start()
        x_cp.start()
        my = lax.axis_index("i")
        left = lax.rem(my + N_DEV - 1, N_DEV)
        right = lax.rem(my + 1, N_DEV)
        diag = lax.rem(my + 2, N_DEV)

        barrier_sem = pltpu.get_barrier_semaphore()
        for nbr in (left, right, diag):
            pl.semaphore_signal(
                barrier_sem, inc=1,
                device_id=(nbr,), device_id_type=pl.DeviceIdType.MESH,
            )
        pl.semaphore_wait(barrier_sem, N_DEV - 1)

        kq = kh // 2

        def w_slice(c, half):
            return pl.ds(c * k + half * kh, kh)

        def wq_slice(c, half, q):
            return pl.ds(c * k + half * kh + q * kq, kq)

        def rdma(src, dst, ssem, rsem, dev):
            return pltpu.make_async_remote_copy(
                src_ref=src, dst_ref=dst, send_sem=ssem, recv_sem=rsem,
                device_id=(dev,), device_id_type=pl.DeviceIdType.MESH,
            )

        ct = 256
        w_cp.wait()
        for t in range(k // ct):
            r = pl.ds(t * ct, ct)
            w8s[r, :] = w_ref[r, :].astype(F8)
        def w_rdma(c, half, q, hop, dev):
            sems = (cws_sems, cwr_sems) if half == 0 else (ccws_sems, ccwr_sems)
            return rdma(
                (w8s if hop == 0 else w_all).at[
                    (pl.ds(half * kh + q * kq, kq) if hop == 0
                     else wq_slice(c, half, q)), :],
                w_all.at[wq_slice(c, half, q), :],
                sems[0].at[2 * hop + q], sems[1].at[2 * hop + q], dev)

        cw0 = [w_rdma(my, 0, q, 0, right) for q in range(2)]
        ccw0 = [w_rdma(my, 1, q, 0, left) for q in range(2)]
        for d in cw0 + ccw0:
            d.start()

        x_cp.wait()
        for t in range(m // ct):
            r = pl.ds(t * ct, ct)
            x8s[r, :] = x_ref[r, :].astype(F8)
        xd2 = rdma(x8s.at[pl.ds(diag * m_out, m_out), :], xg.at[2],
                   xs_sems.at[2], xr_sems.at[2], diag)
        xd0 = rdma(x8s.at[pl.ds(right * m_out, m_out), :], xg.at[0],
                   xs_sems.at[0], xr_sems.at[0], right)
        xd1 = rdma(x8s.at[pl.ds(left * m_out, m_out), :], xg.at[1],
                   xs_sems.at[1], xr_sems.at[1], left)
        for d in (xd2, xd0, xd1):
            d.start()

        mt = 256
        n_tiles = m_out // mt

        def add_gathered_panel(xslot, c, half):
            wv = w_all[w_slice(c, half), :]
            for t in range(n_tiles):
                r = pl.ds(t * mt, mt)
                xv = xg[xslot, t * mt:(t + 1) * mt, half * kh:(half + 1) * kh]
                out_ref[r, :] = out_ref[r, :] + jnp.dot(
                    xv, wv, preferred_element_type=jnp.float32)

        wv0 = w8s[:, :]
        for t in range(n_tiles):
            r = pl.ds(my * m_out + t * mt, mt)
            out_ref[pl.ds(t * mt, mt), :] = jnp.dot(
                x8s[r, :], wv0, preferred_element_type=jnp.float32)

        cw1 = [w_rdma(left, 0, q, 1, right) for q in range(2)]
        ccw1 = [w_rdma(right, 1, q, 1, left) for q in range(2)]
        cw2 = [w_rdma(diag, 0, q, 2, right) for q in range(2)]
        ccw2 = [w_rdma(diag, 1, q, 2, left) for q in range(2)]
        for q in range(2):
            cw0[q].wait_recv()
            cw1[q].start()
            ccw0[q].wait_recv()
            ccw1[q].start()
        for q in range(2):
            cw1[q].wait_recv()
            cw2[q].start()
            ccw1[q].wait_recv()
            ccw2[q].start()

        xd0.wait_recv()
        add_gathered_panel(0, left, 0)
        xd1.wait_recv()
        add_gathered_panel(1, right, 1)
        xd2.wait_recv()
        add_gathered_panel(2, diag, 0)
        add_gathered_panel(2, diag, 1)

        def add_quarter_panel(xslot, c, half, q):
            wv = w_all[wq_slice(c, half, q), :]
            for t in range(n_tiles):
                r = pl.ds(t * mt, mt)
                xv = xg[xslot, t * mt:(t + 1) * mt,
                        half * kh + q * kq:half * kh + (q + 1) * kq]
                out_ref[r, :] = out_ref[r, :] + jnp.dot(
                    xv, wv, preferred_element_type=jnp.float32)

        cw2[0].wait_recv()
        add_quarter_panel(1, right, 0, 0)
        ccw2[0].wait_recv()
        add_quarter_panel(0, left, 1, 0)
        cw2[1].wait_recv()
        add_quarter_panel(1, right, 0, 1)

        for d in [xd0, xd1, xd2] + cw0 + ccw0 + cw1 + ccw1 + cw2:
            d.wait_send()

        ccw2[1].wait_recv()
        sc = s_ref[0, 0]
        wv_last = w_all[wq_slice(left, 1, 1), :]
        for t in range(n_tiles):
            r = pl.ds(t * mt, mt)
            xv = xg[0, t * mt:(t + 1) * mt, kh + kq:kh + 2 * kq]
            y = (out_ref[r, :] + jnp.dot(
                xv, wv_last, preferred_element_type=jnp.float32)) * sc
            out_ref[r, :] = y * jax.nn.sigmoid(y)
        for d in ccw2:
            d.wait_send()

        def _exit_barrier(exit_sem):
            for nbr in (left, right, diag):
                pl.semaphore_signal(
                    exit_sem, inc=1,
                    device_id=(nbr,), device_id_type=pl.DeviceIdType.MESH,
                )
            pl.semaphore_wait(exit_sem, N_DEV - 1)

        pl.run_scoped(_exit_barrier, exit_sem=pltpu.SemaphoreType.REGULAR)

    return pl.pallas_call(
        body,
        out_shape=jax.ShapeDtypeStruct((m_out, n), jnp.float32),
        in_specs=[
            pl.BlockSpec(memory_space=pltpu.MemorySpace.HBM),
            pl.BlockSpec(memory_space=pltpu.MemorySpace.HBM),
            pl.BlockSpec(memory_space=pltpu.SMEM),
        ],
        out_specs=pl.BlockSpec(memory_space=pltpu.VMEM),
        scratch_shapes=[
            pltpu.VMEM((N_DEV * k, n), F8),
            pltpu.VMEM((3, m_out, k), F8),
            pltpu.VMEM((m, k), F8),
            pltpu.VMEM((k, n), F8),
            pltpu.VMEM((m, k), jnp.float32),
            pltpu.VMEM((k, n), jnp.float32),
            pltpu.SemaphoreType.DMA((2,)),
            pltpu.SemaphoreType.DMA((3,)),
            pltpu.SemaphoreType.DMA((3,)),
            pltpu.SemaphoreType.DMA((6,)),
            pltpu.SemaphoreType.DMA((6,)),
            pltpu.SemaphoreType.DMA((6,)),
            pltpu.SemaphoreType.DMA((6,)),
        ],
        compiler_params=pltpu.CompilerParams(
            collective_id=0,
            vmem_limit_bytes=100 * 1024 * 1024,
        ),
    )(x, w_mat, s)
